# baseline (speedup 1.0000x reference)
import functools

import jax
import jax.numpy as jnp
from jax import lax
from jax.experimental import pallas as pl
from jax.experimental.pallas import tpu as pltpu
from jax.experimental.pallas import tpu_sc as plsc

_DPAD = 304
_LPAD = 56


@functools.lru_cache(maxsize=None)
def _make_lookup(b: int, l: int, dim: int):
    info = plsc.get_sparse_core_info()
    nc = info.num_cores
    nw = nc * info.num_subcores
    per_w = b // nw
    nvec = dim // 16

    mesh = plsc.VectorSubcoreMesh(core_axis_name="c", subcore_axis_name="s")

    @functools.partial(
        pl.kernel,
        mesh=mesh,
        compiler_params=pltpu.CompilerParams(use_tc_tiling_on_sc=False),
        out_type=jax.ShapeDtypeStruct((b, l, dim), jnp.float32),
        scratch_types=[
            pltpu.VMEM((per_w, _LPAD), jnp.int32),
            pltpu.VMEM((_LPAD, _DPAD), jnp.float32),
            pltpu.VMEM((_LPAD, _DPAD), jnp.float32),
            pltpu.VMEM((l, dim), jnp.float32),
            pltpu.VMEM((l, dim), jnp.float32),
            pltpu.SemaphoreType.DMA,
            pltpu.SemaphoreType.DMA,
            pltpu.SemaphoreType.DMA,
            pltpu.SemaphoreType.DMA,
        ],
    )
    def lookup_kernel(table_hbm, idx_hbm, out_hbm,
                      idx_v, rows0, rows1, comp0, comp1,
                      sg0, sg1, so0, so1):
        wid = lax.axis_index("s") * nc + lax.axis_index("c")
        base = wid * per_w
        rows = (rows0, rows1)
        comp = (comp0, comp1)
        sg = (sg0, sg1)
        so = (so0, so1)

        pltpu.sync_copy(idx_hbm.at[pl.ds(base, per_w)], idx_v)

        def compact(rbuf, cbuf):
            def row_body(r, carry):
                for k in range(nvec):
                    cbuf[r, pl.ds(16 * k, 16)] = rbuf[r, pl.ds(16 * k, 16)]
                if dim % 16:
                    cbuf[r, pl.ds(dim - 16, 16)] = rbuf[r, pl.ds(dim - 16, 16)]
                return carry
            lax.fori_loop(0, l, row_body, 0)

        gathers = [None, None]
        outs = [None, None]
        gathers[0] = pltpu.async_copy(
            table_hbm.at[idx_v.at[0]], rows[0], sg[0])
        for j in range(per_w):
            p = j % 2
            gathers[p].wait()
            if j + 1 < per_w:
                gathers[(j + 1) % 2] = pltpu.async_copy(
                    table_hbm.at[idx_v.at[j + 1]], rows[(j + 1) % 2],
                    sg[(j + 1) % 2])
            if outs[p] is not None:
                outs[p].wait()
            compact(rows[p], comp[p])
            outs[p] = pltpu.async_copy(comp[p], out_hbm.at[base + j], so[p])
        for o in outs:
            if o is not None:
                o.wait()

    return lookup_kernel


def kernel(table, word_ids):
    b, l = word_ids.shape
    vocab, dim = table.shape
    idx = jnp.pad(word_ids.astype(jnp.int32), ((0, 0), (0, _LPAD - l)))
    tpad = jnp.pad(table, ((0, 0), (0, _DPAD - dim)))
    return _make_lookup(b, l, dim)(tpad, idx)


# speedup vs baseline: 1.5680x; 1.5680x over previous
import functools

import jax
import jax.numpy as jnp
from jax import lax
from jax.experimental import pallas as pl
from jax.experimental.pallas import tpu as pltpu
from jax.experimental.pallas import tpu_sc as plsc

_DPAD = 384
_NT = _DPAD // 128
_CHUNK = 80


@functools.lru_cache(maxsize=None)
def _make_gather(n_total: int):
    info = plsc.get_sparse_core_info()
    nc = info.num_cores
    nw = nc * info.num_subcores
    per_w = n_total // nw
    n_chunks = per_w // _CHUNK

    mesh = plsc.VectorSubcoreMesh(core_axis_name="c", subcore_axis_name="s")

    @functools.partial(
        pl.kernel,
        mesh=mesh,
        out_type=jax.ShapeDtypeStruct((_NT * n_total, 128), jnp.float32),
        scratch_types=[
            pltpu.VMEM((per_w,), jnp.int32),
            pltpu.VMEM((2, _NT, _CHUNK), jnp.int32),
            pltpu.VMEM((2, _NT, _CHUNK, 128), jnp.float32),
            pltpu.SemaphoreType.DMA,
            pltpu.SemaphoreType.DMA,
            pltpu.SemaphoreType.DMA,
            pltpu.SemaphoreType.DMA,
        ],
    )
    def gather_kernel(table_hbm, idx_hbm, out_hbm, idx_v, jb, rows,
                      sg0, sg1, so0, so1):
        wid = lax.axis_index("s") * nc + lax.axis_index("c")
        base = wid * per_w
        sg = (sg0, sg1)
        so = (so0, so1)

        pltpu.sync_copy(idx_hbm.at[pl.ds(base, per_w)], idx_v)

        def build(c, p):
            for k in range(_CHUNK // 16):
                ids = idx_v[pl.ds(c * _CHUNK + 16 * k, 16)]
                v3 = ids * _NT
                for t in range(_NT):
                    jb[p, t, pl.ds(16 * k, 16)] = v3 + t

        def fire(c, p):
            return [
                pltpu.async_copy(table_hbm.at[jb.at[p, t]],
                                 rows.at[p, t], sg[p])
                for t in range(_NT)
            ]

        gathers = [None, None]
        outs = [None, None]
        build(0, 0)
        gathers[0] = fire(0, 0)
        for c in range(n_chunks):
            p = c % 2
            q = (c + 1) % 2
            if c + 1 < n_chunks:
                build(c + 1, q)
            for h in gathers[p]:
                h.wait()
            if c + 1 < n_chunks:
                if outs[q] is not None:
                    for h in outs[q]:
                        h.wait()
                gathers[q] = fire(c + 1, q)
            outs[p] = [
                pltpu.async_copy(
                    rows.at[p, t],
                    out_hbm.at[pl.ds(t * n_total + base + c * _CHUNK, _CHUNK)],
                    so[p])
                for t in range(_NT)
            ]
        for hs in outs:
            if hs is not None:
                for h in hs:
                    h.wait()

    return gather_kernel


_GRP = 4


def _conv_body(p0, p1, p2, out_ref, l, dim):
    planes = (p0, p1, p2)
    for t in range(_NT):
        w = min(dim - 128 * t, 128)
        if w <= 0:
            break
        x = planes[t][...]
        for g in range(_GRP):
            out_ref[g, :, pl.ds(128 * t, w)] = x[l * g:l * g + l, :w]


@functools.lru_cache(maxsize=None)
def _make_convert(b: int, l: int, dim: int, n_total: int):
    blocks_per_plane = n_total // (l * _GRP)

    def plane_spec(t):
        return pl.BlockSpec((l * _GRP, 128),
                            lambda i, t=t: (t * blocks_per_plane + i, 0))

    return pl.pallas_call(
        functools.partial(_conv_body, l=l, dim=dim),
        grid=(b // _GRP,),
        in_specs=[plane_spec(t) for t in range(_NT)],
        out_specs=pl.BlockSpec((_GRP, l, dim), lambda i: (i, 0, 0)),
        out_shape=jax.ShapeDtypeStruct((b, l, dim), jnp.float32),
    )


def kernel(table, word_ids):
    b, l = word_ids.shape
    vocab, dim = table.shape
    idx = word_ids.reshape(-1).astype(jnp.int32)
    t3 = jnp.pad(table, ((0, 0), (0, _DPAD - dim))).reshape(_NT * vocab, 128)
    staged = _make_gather(b * l)(t3, idx)
    return _make_convert(b, l, dim, b * l)(staged, staged, staged)
